# baseline (speedup 1.0000x reference)
import functools

import jax
import jax.numpy as jnp
from jax import lax
from jax.experimental import pallas as pl
from jax.experimental.pallas import tpu as pltpu
from jax.experimental.pallas import tpu_sc as plsc

_FEAT_STRIDE = 16.0
_SCORE_THRESH = 0.7
_DIST_THRESH = 16.0
_MAX_OUT = 50
_OUT_PAD = 64
_NEG = -3.0e38
_SENT = 1.0e30


def _worker_impl(offv, maxwv,
                 ms_v, mp_v, p0_v, p1_v, l1_v, l2_v, kept_v,
                 bp0_v, bp1_v, bs_v, bv_v, col_v, cls_v):
    FW = ms_v.shape[0]
    nchunk = FW // 16
    nsup = nchunk // 16
    lane = lax.broadcasted_iota(jnp.int32, (16,), 0)

    def _clamp(p):
        p = jnp.where(p < 0.0, 0.0, p)
        p = jnp.where(p > maxwv, maxwv, p)
        return p

    def prolog(i, carry):
        base = i * 16
        sl = pl.ds(base, 16)
        gif = (lane + base).astype(jnp.float32)
        center = (gif + 0.5) * _FEAT_STRIDE + offv
        p0c = _clamp(p0_v[sl] * _FEAT_STRIDE + center)
        p1c = _clamp(p1_v[sl] * _FEAT_STRIDE + center)
        p0_v[sl] = p0c
        p1_v[sl] = p1c
        mp_v[sl] = (p0c + p1c) * 0.5
        cm = jnp.max(ms_v[sl])
        jc = i // 16
        il = i % 16
        l1sl = pl.ds(jc * 16, 16)
        l1c = l1_v[l1sl]
        l1_v[l1sl] = jnp.where(lane == il, cm, l1c)
        cls_v[sl] = jnp.zeros((16,), jnp.float32)
        return carry

    lax.fori_loop(0, nchunk, prolog, 0)

    l2_v[pl.ds(0, 16)] = jnp.full((16,), _NEG, jnp.float32)
    l2_v[pl.ds(16, 16)] = jnp.full((16,), _NEG, jnp.float32)

    def build_l2(j, carry):
        sm = jnp.max(l1_v[pl.ds(j * 16, 16)])
        jj = j // 16
        jl = j % 16
        sl = pl.ds(jj * 16, 16)
        l2c = l2_v[sl]
        l2_v[sl] = jnp.where(lane == jl, sm, l2c)
        return carry

    lax.fori_loop(0, nsup, build_l2, 0)

    zero16 = jnp.zeros((16,), jnp.float32)
    for q in range(4):
        sl = pl.ds(q * 16, 16)
        kept_v[sl] = jnp.full((16,), _SENT, jnp.float32)
        bp0_v[sl] = zero16
        bp1_v[sl] = zero16
        bs_v[sl] = zero16
        bv_v[sl] = zero16
        col_v[sl] = jnp.full((16,), -1, jnp.int32)

    def cond(carry):
        kc, cont = carry
        return cont > 0

    def body(carry):
        kc, _ = carry
        v0 = l2_v[pl.ds(0, 16)]
        v1 = l2_v[pl.ds(16, 16)]
        upd = v1 >= v0
        accv = jnp.where(upd, v1, v0)
        accj = jnp.where(upd, lane + 16, lane)
        smax = jnp.max(accv)
        jsel = jnp.max(jnp.where(accv == smax, accj, -1))
        l1sl = pl.ds(jsel * 16, 16)
        l1v = l1_v[l1sl]
        cmax = jnp.max(l1v)
        clane = jnp.max(jnp.where(l1v == cmax, lane, -1))
        ch = jsel * 16 + clane
        esl = pl.ds(ch * 16, 16)
        msv = ms_v[esl]
        emax = jnp.max(msv)
        lsel = jnp.max(jnp.where(msv == emax, lane, -1))
        valid = emax >= _SCORE_THRESH
        sel1 = lane == lsel
        m_e = jnp.max(jnp.where(sel1, mp_v[esl], _NEG))
        p0_e = jnp.max(jnp.where(sel1, p0_v[esl], _NEG))
        p1_e = jnp.max(jnp.where(sel1, p1_v[esl], _NEG))
        sflag = jnp.float32(0.0)
        for q in range(4):
            kq = kept_v[pl.ds(q * 16, 16)]
            sflag = jnp.maximum(
                sflag,
                jnp.max(jnp.where(jnp.abs(kq - m_e) <= _DIST_THRESH, 1.0, 0.0)))
        supp = sflag > 0.5
        dval = jnp.where(valid, jnp.float32(-1.0), emax)
        msv2 = jnp.where(sel1, dval, msv)
        ms_v[esl] = msv2
        ncm = jnp.max(msv2)
        l1v2 = jnp.where(lane == clane, ncm, l1v)
        l1_v[l1sl] = l1v2
        nsm = jnp.max(l1v2)
        jj = jsel // 16
        jl = jsel % 16
        l2sl = pl.ds(jj * 16, 16)
        l2c = l2_v[l2sl]
        l2_v[l2sl] = jnp.where(lane == jl, nsm, l2c)
        do_keep = valid & jnp.logical_not(supp)
        kchunk = kc // 16
        klane = kc % 16
        ksl = pl.ds(kchunk * 16, 16)
        ksel = lane == klane
        kept_v[ksl] = jnp.where(ksel, jnp.where(do_keep, m_e, _SENT), kept_v[ksl])
        bp0_v[ksl] = jnp.where(ksel, jnp.where(do_keep, p0_e, 0.0), bp0_v[ksl])
        bp1_v[ksl] = jnp.where(ksel, jnp.where(do_keep, p1_e, 0.0), bp1_v[ksl])
        bs_v[ksl] = jnp.where(ksel, jnp.where(do_keep, emax, 0.0), bs_v[ksl])
        bv_v[ksl] = jnp.where(ksel, jnp.where(do_keep, 1.0, 0.0), bv_v[ksl])
        col_f = m_e * (1.0 / _FEAT_STRIDE)
        col_i = col_f.astype(jnp.int32)
        col_i = col_i - jnp.where(col_i.astype(jnp.float32) > col_f, 1, 0)
        colk = jnp.where(m_e < 0.0, jnp.int32(-1), col_i)
        col_v[ksl] = jnp.where(ksel, jnp.where(do_keep, colk, -1), col_v[ksl])
        kc2 = kc + jnp.where(do_keep, 1, 0)
        cont = jnp.where(valid & (kc2 < _MAX_OUT), 1, 0)
        return (kc2, cont)

    kc_fin, _ = lax.while_loop(cond, body, (jnp.int32(0), jnp.int32(1)))

    def cls_scatter(k, carry):
        @pl.when(k < kc_fin)
        def _():
            kchunk = k // 16
            klane = k % 16
            colv = col_v[pl.ds(kchunk * 16, 16)]
            col = jnp.max(jnp.where(lane == klane, colv, -2))

            @pl.when(col >= 0)
            def _():
                cc = col // 16
                cl = col % 16
                sl = pl.ds(cc * 16, 16)
                cv = cls_v[sl]
                cls_v[sl] = jnp.where(lane == cl, cv + 1.0, cv)
        return carry

    lax.fori_loop(0, _MAX_OUT, cls_scatter, 0)


def _sc_body(scores_hbm, d0_hbm, d1_hbm, off_hbm, maxw_hbm,
             p0_hbm, p1_hbm, s_out_hbm, v_hbm, cls_hbm,
             ms_v, mp_v, p0_v, p1_v, l1_v, l2_v, kept_v,
             bp0_v, bp1_v, bs_v, bv_v, col_v, cls_v, off_v, maxw_v):
    c_id = lax.axis_index("c")
    s_id = lax.axis_index("s")
    row = s_id
    B = scores_hbm.shape[0]

    @pl.when((c_id == 0) & (s_id < B))
    def _worker():
        pltpu.sync_copy(scores_hbm.at[row], ms_v)
        pltpu.sync_copy(d0_hbm.at[row], p0_v)
        pltpu.sync_copy(d1_hbm.at[row], p1_v)
        pltpu.sync_copy(off_hbm.at[row], off_v)
        pltpu.sync_copy(maxw_hbm.at[row], maxw_v)
        _worker_impl(off_v[...], maxw_v[...],
                     ms_v, mp_v, p0_v, p1_v, l1_v, l2_v, kept_v,
                     bp0_v, bp1_v, bs_v, bv_v, col_v, cls_v)
        pltpu.sync_copy(bp0_v, p0_hbm.at[row])
        pltpu.sync_copy(bp1_v, p1_hbm.at[row])
        pltpu.sync_copy(bs_v, s_out_hbm.at[row])
        pltpu.sync_copy(bv_v, v_hbm.at[row])
        pltpu.sync_copy(cls_v, cls_hbm.at[row])


@functools.lru_cache(maxsize=None)
def _make_sc(B, FW):
    mesh = plsc.VectorSubcoreMesh(core_axis_name="c", subcore_axis_name="s",
                                  num_cores=2, num_subcores=16)
    f32 = jnp.float32
    sc_call = pl.kernel(
        _sc_body,
        out_type=[
            jax.ShapeDtypeStruct((B, _OUT_PAD), f32),
            jax.ShapeDtypeStruct((B, _OUT_PAD), f32),
            jax.ShapeDtypeStruct((B, _OUT_PAD), f32),
            jax.ShapeDtypeStruct((B, _OUT_PAD), f32),
            jax.ShapeDtypeStruct((B, FW), f32),
        ],
        mesh=mesh,
        scratch_types=[
            pltpu.VMEM((FW,), f32),
            pltpu.VMEM((FW,), f32),
            pltpu.VMEM((FW,), f32),
            pltpu.VMEM((FW,), f32),
            pltpu.VMEM((FW // 16,), f32),
            pltpu.VMEM((32,), f32),
            pltpu.VMEM((_OUT_PAD,), f32),
            pltpu.VMEM((_OUT_PAD,), f32),
            pltpu.VMEM((_OUT_PAD,), f32),
            pltpu.VMEM((_OUT_PAD,), f32),
            pltpu.VMEM((_OUT_PAD,), f32),
            pltpu.VMEM((_OUT_PAD,), jnp.int32),
            pltpu.VMEM((FW,), f32),
            pltpu.VMEM((16,), f32),
            pltpu.VMEM((16,), f32),
        ],
        compiler_params=pltpu.CompilerParams(needs_layout_passes=False),
    )
    return jax.jit(sc_call)


def kernel(pred_cls_logit, pred_delta, img_width, real_images_width):
    B, FW = pred_cls_logit.shape
    scores = jax.nn.sigmoid(pred_cls_logit)
    d0 = pred_delta[..., 0]
    d1 = pred_delta[..., 1]
    off = (jnp.asarray(img_width) - FW * 16).astype(jnp.float32)
    off_b = jnp.broadcast_to(jnp.reshape(off, (1, 1)), (B, 16))
    maxw_b = jnp.broadcast_to(
        (jnp.asarray(real_images_width, jnp.float32) - 1.0).reshape(B, 1), (B, 16))
    P0, P1, S, V, cls = _make_sc(B, FW)(scores, d0, d1, off_b, maxw_b)
    P0, P1, S, V = (a[:, :_MAX_OUT] for a in (P0, P1, S, V))
    nms_positions = jnp.stack([P0, P1, V], axis=-1)
    nms_scores = jnp.stack([S, V], axis=-1)
    return nms_positions, nms_scores, cls


# speedup vs baseline: 1.0993x; 1.0993x over previous
import functools

import jax
import jax.numpy as jnp
from jax import lax
from jax.experimental import pallas as pl
from jax.experimental.pallas import tpu as pltpu
from jax.experimental.pallas import tpu_sc as plsc

_FEAT_STRIDE = 16.0
_SCORE_THRESH = 0.7
_DIST_THRESH = 16.0
_MAX_OUT = 50
_OUT_PAD = 64
_NEG = -3.0e38
_SENT = 1.0e30


def _worker_impl(offv, maxwv,
                 ms_v, mp_v, p0_v, p1_v, l1_v, l2_v, kept_v,
                 bp0_v, bp1_v, bs_v, bv_v, cls_v):
    FW = ms_v.shape[0]
    nchunk = FW // 16
    nsup = nchunk // 16
    lane = lax.broadcasted_iota(jnp.int32, (16,), 0)

    def _clamp(p):
        p = jnp.where(p < 0.0, 0.0, p)
        p = jnp.where(p > maxwv, maxwv, p)
        return p

    def prolog(g, carry):
        base0 = g * 256
        acc = jnp.full((16,), _NEG, jnp.float32)
        for il in range(16):
            base = base0 + il * 16
            sl = pl.ds(base, 16)
            gif = (lane + base).astype(jnp.float32)
            center = (gif + 0.5) * _FEAT_STRIDE + offv
            p0c = _clamp(p0_v[sl] * _FEAT_STRIDE + center)
            p1c = _clamp(p1_v[sl] * _FEAT_STRIDE + center)
            p0_v[sl] = p0c
            p1_v[sl] = p1c
            mp_v[sl] = (p0c + p1c) * 0.5
            cls_v[sl] = jnp.zeros((16,), jnp.float32)
            cm = jnp.max(ms_v[sl])
            acc = jnp.where(lane == il, cm, acc)
        l1_v[pl.ds(g * 16, 16)] = acc
        return carry

    lax.fori_loop(0, nsup, prolog, 0)

    acc0 = jnp.full((16,), _NEG, jnp.float32)
    acc1 = jnp.full((16,), _NEG, jnp.float32)
    for j in range(nsup):
        sm = jnp.max(l1_v[pl.ds(j * 16, 16)])
        if j < 16:
            acc0 = jnp.where(lane == j, sm, acc0)
        else:
            acc1 = jnp.where(lane == (j - 16), sm, acc1)
    l2_v[pl.ds(0, 16)] = acc0
    l2_v[pl.ds(16, 16)] = acc1

    zero16 = jnp.zeros((16,), jnp.float32)
    for q in range(4):
        sl = pl.ds(q * 16, 16)
        kept_v[sl] = jnp.full((16,), _SENT, jnp.float32)
        bp0_v[sl] = zero16
        bp1_v[sl] = zero16
        bs_v[sl] = zero16
        bv_v[sl] = zero16

    def cond(carry):
        kc, cont = carry
        return cont > 0

    def body(carry):
        kc, _ = carry
        v0 = l2_v[pl.ds(0, 16)]
        v1 = l2_v[pl.ds(16, 16)]
        upd = v1 >= v0
        accv = jnp.where(upd, v1, v0)
        accj = jnp.where(upd, lane + 16, lane)
        smax = jnp.max(accv)
        jsel = jnp.max(jnp.where(accv == smax, accj, -1))
        l1sl = pl.ds(jsel * 16, 16)
        l1v = l1_v[l1sl]
        clane = jnp.max(jnp.where(l1v == smax, lane, -1))
        ch = jsel * 16 + clane
        esl = pl.ds(ch * 16, 16)
        msv = ms_v[esl]
        lsel = jnp.max(jnp.where(msv == smax, lane, -1))
        valid = smax >= _SCORE_THRESH
        sel1 = lane == lsel
        m_e = jnp.max(jnp.where(sel1, mp_v[esl], _NEG))
        p0_e = jnp.max(jnp.where(sel1, p0_v[esl], _NEG))
        p1_e = jnp.max(jnp.where(sel1, p1_v[esl], _NEG))
        d01 = jnp.minimum(jnp.abs(kept_v[pl.ds(0, 16)] - m_e),
                          jnp.abs(kept_v[pl.ds(16, 16)] - m_e))
        d23 = jnp.minimum(jnp.abs(kept_v[pl.ds(32, 16)] - m_e),
                          jnp.abs(kept_v[pl.ds(48, 16)] - m_e))
        supp = jnp.min(jnp.minimum(d01, d23)) <= _DIST_THRESH
        dval = jnp.where(valid, jnp.float32(-1.0), smax)
        msv2 = jnp.where(sel1, dval, msv)
        ms_v[esl] = msv2
        ncm = jnp.max(msv2)
        l1v2 = jnp.where(lane == clane, ncm, l1v)
        l1_v[l1sl] = l1v2
        nsm = jnp.max(l1v2)
        jj = jsel // 16
        jl = jsel % 16
        l2sl = pl.ds(jj * 16, 16)
        l2c = l2_v[l2sl]
        l2_v[l2sl] = jnp.where(lane == jl, nsm, l2c)
        do_keep = valid & jnp.logical_not(supp)
        kchunk = kc // 16
        klane = kc % 16
        ksl = pl.ds(kchunk * 16, 16)
        ksel = lane == klane
        kept_v[ksl] = jnp.where(ksel, jnp.where(do_keep, m_e, _SENT), kept_v[ksl])
        bp0_v[ksl] = jnp.where(ksel, jnp.where(do_keep, p0_e, 0.0), bp0_v[ksl])
        bp1_v[ksl] = jnp.where(ksel, jnp.where(do_keep, p1_e, 0.0), bp1_v[ksl])
        bs_v[ksl] = jnp.where(ksel, jnp.where(do_keep, smax, 0.0), bs_v[ksl])
        bv_v[ksl] = jnp.where(ksel, jnp.where(do_keep, 1.0, 0.0), bv_v[ksl])
        col_f = m_e * (1.0 / _FEAT_STRIDE)
        col_i = col_f.astype(jnp.int32)
        col_i = col_i - jnp.where(col_i.astype(jnp.float32) > col_f, 1, 0)
        colk = jnp.where(m_e < 0.0, jnp.int32(-1), col_i)
        cc = jnp.maximum(colk, 0) // 16
        cl = colk - cc * 16
        gate = jnp.where(do_keep, jnp.float32(1.0), jnp.float32(0.0))
        csl = pl.ds(cc * 16, 16)
        cv = cls_v[csl]
        cls_v[csl] = cv + jnp.where(lane == cl, gate, 0.0)
        kc2 = kc + jnp.where(do_keep, 1, 0)
        cont = jnp.where(valid & (kc2 < _MAX_OUT), 1, 0)
        return (kc2, cont)

    lax.while_loop(cond, body, (jnp.int32(0), jnp.int32(1)))


def _sc_body(scores_hbm, d0_hbm, d1_hbm, off_hbm, maxw_hbm,
             p0_hbm, p1_hbm, s_out_hbm, v_hbm, cls_hbm,
             ms_v, mp_v, p0_v, p1_v, l1_v, l2_v, kept_v,
             bp0_v, bp1_v, bs_v, bv_v, cls_v, off_v, maxw_v):
    c_id = lax.axis_index("c")
    s_id = lax.axis_index("s")
    row = s_id
    B = scores_hbm.shape[0]

    @pl.when((c_id == 0) & (s_id < B))
    def _worker():
        pltpu.sync_copy(scores_hbm.at[row], ms_v)
        pltpu.sync_copy(d0_hbm.at[row], p0_v)
        pltpu.sync_copy(d1_hbm.at[row], p1_v)
        pltpu.sync_copy(off_hbm.at[row], off_v)
        pltpu.sync_copy(maxw_hbm.at[row], maxw_v)
        _worker_impl(off_v[...], maxw_v[...],
                     ms_v, mp_v, p0_v, p1_v, l1_v, l2_v, kept_v,
                     bp0_v, bp1_v, bs_v, bv_v, cls_v)
        pltpu.sync_copy(bp0_v, p0_hbm.at[row])
        pltpu.sync_copy(bp1_v, p1_hbm.at[row])
        pltpu.sync_copy(bs_v, s_out_hbm.at[row])
        pltpu.sync_copy(bv_v, v_hbm.at[row])
        pltpu.sync_copy(cls_v, cls_hbm.at[row])


@functools.lru_cache(maxsize=None)
def _make_sc(B, FW):
    mesh = plsc.VectorSubcoreMesh(core_axis_name="c", subcore_axis_name="s",
                                  num_cores=2, num_subcores=16)
    f32 = jnp.float32
    sc_call = pl.kernel(
        _sc_body,
        out_type=[
            jax.ShapeDtypeStruct((B, _OUT_PAD), f32),
            jax.ShapeDtypeStruct((B, _OUT_PAD), f32),
            jax.ShapeDtypeStruct((B, _OUT_PAD), f32),
            jax.ShapeDtypeStruct((B, _OUT_PAD), f32),
            jax.ShapeDtypeStruct((B, FW), f32),
        ],
        mesh=mesh,
        scratch_types=[
            pltpu.VMEM((FW,), f32),
            pltpu.VMEM((FW,), f32),
            pltpu.VMEM((FW,), f32),
            pltpu.VMEM((FW,), f32),
            pltpu.VMEM((FW // 16,), f32),
            pltpu.VMEM((32,), f32),
            pltpu.VMEM((_OUT_PAD,), f32),
            pltpu.VMEM((_OUT_PAD,), f32),
            pltpu.VMEM((_OUT_PAD,), f32),
            pltpu.VMEM((_OUT_PAD,), f32),
            pltpu.VMEM((_OUT_PAD,), f32),
            pltpu.VMEM((FW,), f32),
            pltpu.VMEM((16,), f32),
            pltpu.VMEM((16,), f32),
        ],
        compiler_params=pltpu.CompilerParams(needs_layout_passes=False),
    )
    return jax.jit(sc_call)


def kernel(pred_cls_logit, pred_delta, img_width, real_images_width):
    B, FW = pred_cls_logit.shape
    scores = jax.nn.sigmoid(pred_cls_logit)
    d0 = pred_delta[..., 0]
    d1 = pred_delta[..., 1]
    off = (jnp.asarray(img_width) - FW * 16).astype(jnp.float32)
    off_b = jnp.broadcast_to(jnp.reshape(off, (1, 1)), (B, 16))
    maxw_b = jnp.broadcast_to(
        (jnp.asarray(real_images_width, jnp.float32) - 1.0).reshape(B, 1), (B, 16))
    P0, P1, S, V, cls = _make_sc(B, FW)(scores, d0, d1, off_b, maxw_b)
    P0, P1, S, V = (a[:, :_MAX_OUT] for a in (P0, P1, S, V))
    nms_positions = jnp.stack([P0, P1, V], axis=-1)
    nms_scores = jnp.stack([S, V], axis=-1)
    return nms_positions, nms_scores, cls
